# Initial kernel scaffold; baseline (speedup 1.0000x reference)
import jax
import jax.numpy as jnp
from jax.experimental import pallas as pl


def kernel(x, expert_sel, keys_w, values_w):
    raise NotImplementedError("write your pallas kernel here")


# speedup vs baseline: 2.0141x; 2.0141x over previous
import functools

import jax
import jax.numpy as jnp
from jax.experimental import pallas as pl

DMODEL = 1024
NEXP = 8
ESZ = 128
TOPK = 2


def _moe_tile(x_ref, selt_ref, kflat_ref, vflat_ref, out_ref):
    x = x_ref[...]
    logits = jnp.dot(x, selt_ref[...], preferred_element_type=jnp.float32)
    sel = jax.nn.sigmoid(logits)
    eidx = jax.lax.broadcasted_iota(jnp.int32, sel.shape, 1)
    i1 = jnp.argmax(sel, axis=1)
    m1 = eidx == i1[:, None]
    sel_masked = jnp.where(m1, -jnp.inf, sel)
    i2 = jnp.argmax(sel_masked, axis=1)
    m2 = eidx == i2[:, None]
    gate = jnp.where(m1 | m2, sel, 0.0)

    h = jnp.dot(x, kflat_ref[...], preferred_element_type=jnp.float32)
    h = jax.nn.relu(h)
    h = h.reshape(x.shape[0], NEXP, ESZ) * gate[:, :, None]
    h = h.reshape(x.shape[0], NEXP * ESZ)
    out_ref[...] = jnp.dot(h, vflat_ref[...], preferred_element_type=jnp.float32)


@jax.jit
def kernel(x, expert_sel, keys_w, values_w):
    B, S, D = x.shape
    N = B * S
    xf = x.reshape(N, D)
    selt = expert_sel.T
    kflat = keys_w.transpose(1, 0, 2).reshape(D, NEXP * ESZ)
    vflat = values_w.reshape(NEXP * ESZ, D)

    T = 512
    grid = (N // T,)
    out = pl.pallas_call(
        _moe_tile,
        grid=grid,
        in_specs=[
            pl.BlockSpec((T, D), lambda i: (i, 0)),
            pl.BlockSpec((D, NEXP), lambda i: (0, 0)),
            pl.BlockSpec((D, NEXP * ESZ), lambda i: (0, 0)),
            pl.BlockSpec((NEXP * ESZ, D), lambda i: (0, 0)),
        ],
        out_specs=pl.BlockSpec((T, D), lambda i: (i, 0)),
        out_shape=jax.ShapeDtypeStruct((N, D), jnp.float32),
    )(xf, selt, kflat, vflat)
    return out.reshape(B, S, D)
